# Initial kernel scaffold; baseline (speedup 1.0000x reference)
import jax
import jax.numpy as jnp
from jax.experimental import pallas as pl


def kernel(x, edge_index, W1, b1, W2, b2):
    raise NotImplementedError("write your pallas kernel here")


# speedup vs baseline: 26.6325x; 26.6325x over previous
import functools

import jax
import jax.numpy as jnp
from jax import lax
from jax.experimental import pallas as pl
from jax.experimental.pallas import tpu as pltpu
from jax.experimental.pallas import tpu_sc as plsc

N_REAL = 10000
ROWS = 10240
DUMMY = ROWS - N_REAL
NC = 2
NS = 16
CHUNK = 128
CPT = 80
E_PAD = NC * NS * CPT * CHUNK
RPT = ROWS // NS

D_IN = 128
D_HID = 128
D_CLS = 40
D_CLS_PAD = 64

_MESH = plsc.VectorSubcoreMesh(core_axis_name="c", subcore_axis_name="s")


def _sc_degree(dst_hbm, zeros_hbm, out_hbm, dst_v, ones_v, acc_sh):
    c = lax.axis_index("c")
    s = lax.axis_index("s")
    pltpu.sync_copy(dst_hbm.at[c, s], dst_v)
    sl = pl.ds(s * RPT, RPT)
    pltpu.sync_copy(zeros_hbm.at[sl], acc_sh.at[sl])
    for k in range(CHUNK // 16):
        ones_v[pl.ds(k * 16, 16)] = jnp.ones((16,), jnp.float32)
    plsc.subcore_barrier()

    def body(j, carry):
        pltpu.sync_copy(ones_v, acc_sh.at[dst_v.at[j]], add=True)
        return carry

    lax.fori_loop(0, CPT, body, 0)
    plsc.subcore_barrier()
    pltpu.sync_copy(acc_sh.at[sl], out_hbm.at[c, sl])


_degree_kernel = functools.partial(
    pl.kernel,
    out_type=jax.ShapeDtypeStruct((NC, ROWS), jnp.float32),
    mesh=_MESH,
    scratch_types=[
        pltpu.VMEM((CPT, CHUNK), jnp.int32),
        pltpu.VMEM((CHUNK,), jnp.float32),
        pltpu.VMEM_SHARED((ROWS,), jnp.float32),
    ],
)(_sc_degree)


def _sc_agg(src_hbm, dst_hbm, y_hbm, zeros_hbm, out_hbm,
            src_v, dst_v, rows_v, acc_sh, *sems):
    isem = sems[0:2]
    gsem = sems[2:4]
    ssem = sems[4:6]
    c = lax.axis_index("c")
    s = lax.axis_index("s")
    sl = pl.ds(s * RPT, RPT)
    pltpu.sync_copy(zeros_hbm.at[sl], acc_sh.at[sl])
    plsc.subcore_barrier()

    def i_descs(j, b):
        return (pltpu.make_async_copy(src_hbm.at[c, s, j], src_v.at[b], isem[b]),
                pltpu.make_async_copy(dst_hbm.at[c, s, j], dst_v.at[b], isem[b]))

    def g_desc(j, b):
        del j
        return pltpu.make_async_copy(
            y_hbm.at[src_v.at[b]], rows_v.at[b], gsem[b])

    def s_desc(j, b):
        del j
        return pltpu.make_async_copy(
            rows_v.at[b], acc_sh.at[dst_v.at[b]], ssem[b])

    def start_idx(j, b):
        d1, d2 = i_descs(j, b)
        d1.start()
        d2.start()

    def wait_idx(j, b):
        d1, d2 = i_descs(j, b)
        d1.wait()
        d2.wait()

    for b in range(2):
        start_idx(b, b)
        wait_idx(b, b)
        g_desc(b, b).start()

    def body(i, carry):
        j0 = 2 * i
        for b in range(2):
            g_desc(j0 + b, b).wait()
            s_desc(j0 + b, b).start(add=True)
        for b in range(2):
            s_desc(j0 + b, b).wait()
            start_idx(j0 + 2 + b, b)
            wait_idx(j0 + 2 + b, b)
            g_desc(j0 + 2 + b, b).start()
        return carry

    lax.fori_loop(0, CPT // 2 - 1, body, 0)
    tail = CPT - 2
    for b in range(2):
        g_desc(tail + b, b).wait()
        s_desc(tail + b, b).start(add=True)
    for b in range(2):
        s_desc(tail + b, b).wait()
    plsc.subcore_barrier()
    pltpu.sync_copy(acc_sh.at[sl], out_hbm.at[c, sl])


_agg_kernel = functools.partial(
    pl.kernel,
    out_type=jax.ShapeDtypeStruct((NC, ROWS, D_HID), jnp.float32),
    mesh=_MESH,
    scratch_types=[
        pltpu.VMEM((2, CHUNK), jnp.int32),
        pltpu.VMEM((2, CHUNK), jnp.int32),
        pltpu.VMEM((2, CHUNK, D_HID), jnp.float32),
        pltpu.VMEM_SHARED((ROWS, D_HID), jnp.float32),
    ] + [pltpu.SemaphoreType.DMA] * 6,
)(_sc_agg)


_BLK = 512
_GRID = ROWS // _BLK


def _dis_block(deg_ref, i):
    d = deg_ref[0, pl.ds(i * _BLK, _BLK)] + deg_ref[1, pl.ds(i * _BLK, _BLK)]
    return lax.rsqrt(d + 1.0)[:, None]


def _row_mask(i):
    rows = i * _BLK + lax.broadcasted_iota(jnp.int32, (_BLK, 1), 0)
    return (rows < N_REAL).astype(jnp.float32)


def _tc_mm1(x_ref, w_ref, deg_ref, y_ref):
    i = pl.program_id(0)
    dis = _dis_block(deg_ref, i)
    y_ref[...] = jnp.dot(x_ref[...], w_ref[...],
                         preferred_element_type=jnp.float32) * dis


def _tc_hidden(agg_ref, y1_ref, deg_ref, b1_ref, t_ref):
    i = pl.program_id(0)
    dis = _dis_block(deg_ref, i)
    h = (agg_ref[0] + agg_ref[1] + y1_ref[...]) * dis + b1_ref[...]
    h = jnp.maximum(h, 0.0)
    t_ref[...] = h * dis * _row_mask(i)


def _tc_out(agg_ref, t_ref, deg_ref, w2_ref, b2_ref, o_ref):
    i = pl.program_id(0)
    dis = _dis_block(deg_ref, i)
    ah = (agg_ref[0] + agg_ref[1] + t_ref[...]) * dis
    z = jnp.dot(ah, w2_ref[...],
                preferred_element_type=jnp.float32)[:, :D_CLS] + b2_ref[...]
    m = jnp.max(z, axis=1, keepdims=True)
    lse = jnp.log(jnp.sum(jnp.exp(z - m), axis=1, keepdims=True)) + m
    o_ref[...] = z - lse


def _mm1_call(x_pad, W1, degp):
    return pl.pallas_call(
        _tc_mm1,
        grid=(_GRID,),
        in_specs=[
            pl.BlockSpec((_BLK, D_IN), lambda i: (i, 0)),
            pl.BlockSpec((D_IN, D_HID), lambda i: (0, 0)),
            pl.BlockSpec((NC, ROWS), lambda i: (0, 0)),
        ],
        out_specs=pl.BlockSpec((_BLK, D_HID), lambda i: (i, 0)),
        out_shape=jax.ShapeDtypeStruct((ROWS, D_HID), jnp.float32),
    )(x_pad, W1, degp)


def _hidden_call(agg1, y1, degp, b1):
    return pl.pallas_call(
        _tc_hidden,
        grid=(_GRID,),
        in_specs=[
            pl.BlockSpec((NC, _BLK, D_HID), lambda i: (0, i, 0)),
            pl.BlockSpec((_BLK, D_HID), lambda i: (i, 0)),
            pl.BlockSpec((NC, ROWS), lambda i: (0, 0)),
            pl.BlockSpec((1, D_HID), lambda i: (0, 0)),
        ],
        out_specs=pl.BlockSpec((_BLK, D_HID), lambda i: (i, 0)),
        out_shape=jax.ShapeDtypeStruct((ROWS, D_HID), jnp.float32),
    )(agg1, y1, degp, b1)


def _out_call(agg2, t, degp, W2p, b2):
    return pl.pallas_call(
        _tc_out,
        grid=(_GRID,),
        in_specs=[
            pl.BlockSpec((NC, _BLK, D_HID), lambda i: (0, i, 0)),
            pl.BlockSpec((_BLK, D_HID), lambda i: (i, 0)),
            pl.BlockSpec((NC, ROWS), lambda i: (0, 0)),
            pl.BlockSpec((D_HID, D_CLS_PAD), lambda i: (0, 0)),
            pl.BlockSpec((1, D_CLS), lambda i: (0, 0)),
        ],
        out_specs=pl.BlockSpec((_BLK, D_CLS), lambda i: (i, 0)),
        out_shape=jax.ShapeDtypeStruct((ROWS, D_CLS), jnp.float32),
    )(agg2, t, degp, W2p, b2)


def kernel(x, edge_index, W1, b1, W2, b2):
    ei = edge_index.astype(jnp.int32)
    src = ei[0]
    dst = ei[1]
    pad_n = E_PAD - src.shape[0]
    pad_iota = jnp.arange(pad_n, dtype=jnp.int32)
    pad_rows = N_REAL + pad_iota % DUMMY
    srcp = jnp.concatenate([src, pad_rows]).reshape(NC, NS, CPT, CHUNK)
    dstp = jnp.concatenate([dst, pad_rows]).reshape(NC, NS, CPT, CHUNK)

    zeros_deg = jnp.zeros((ROWS,), jnp.float32)
    zeros128 = jnp.zeros((ROWS, D_HID), jnp.float32)
    x_pad = jnp.concatenate([x, jnp.zeros((DUMMY, D_IN), jnp.float32)])
    W2p = jnp.concatenate(
        [W2, jnp.zeros((D_HID, D_CLS_PAD - D_CLS), jnp.float32)], axis=1)

    degp = _degree_kernel(dstp, zeros_deg)
    y1 = _mm1_call(x_pad, W1, degp)
    agg1 = _agg_kernel(srcp, dstp, y1, zeros128)
    t = _hidden_call(agg1, y1, degp, b1.reshape(1, D_HID))
    agg2 = _agg_kernel(srcp, dstp, t, zeros128)
    out = _out_call(agg2, t, degp, W2p, b2.reshape(1, D_CLS))
    return out[:N_REAL]


# speedup vs baseline: 30.2746x; 1.1368x over previous
import functools

import jax
import jax.numpy as jnp
from jax import lax
from jax.experimental import pallas as pl
from jax.experimental.pallas import tpu as pltpu
from jax.experimental.pallas import tpu_sc as plsc

N_REAL = 10000
ROWS = 10240
DUMMY = ROWS - N_REAL
NC = 2
NS = 16
CHUNK = 128
CPT = 80
E_PAD = NC * NS * CPT * CHUNK
RPT = ROWS // NS

D_IN = 128
D_HID = 128
D_CLS = 40
D_CLS_PAD = 64

_MESH = plsc.VectorSubcoreMesh(core_axis_name="c", subcore_axis_name="s")


def _sc_degree(dst_hbm, zeros_hbm, out_hbm, dst_v, ones_v, acc_sh):
    c = lax.axis_index("c")
    s = lax.axis_index("s")
    pltpu.sync_copy(dst_hbm.at[c, s], dst_v)
    sl = pl.ds(s * RPT, RPT)
    pltpu.sync_copy(zeros_hbm.at[sl], acc_sh.at[sl])
    for k in range(CHUNK // 16):
        ones_v[pl.ds(k * 16, 16)] = jnp.ones((16,), jnp.float32)
    plsc.subcore_barrier()

    def body(j, carry):
        pltpu.sync_copy(ones_v, acc_sh.at[dst_v.at[j]], add=True)
        return carry

    lax.fori_loop(0, CPT, body, 0)
    plsc.subcore_barrier()
    pltpu.sync_copy(acc_sh.at[sl], out_hbm.at[c, sl])


_degree_kernel = functools.partial(
    pl.kernel,
    out_type=jax.ShapeDtypeStruct((NC, ROWS), jnp.float32),
    mesh=_MESH,
    scratch_types=[
        pltpu.VMEM((CPT, CHUNK), jnp.int32),
        pltpu.VMEM((CHUNK,), jnp.float32),
        pltpu.VMEM_SHARED((ROWS,), jnp.float32),
    ],
)(_sc_degree)


def _sc_agg(D, nbuf, cpt, src_hbm, dst_hbm, y_hbm, zeros_hbm, out_hbm,
            src_v, dst_v, rows_v, acc_sh, *sems):
    isem = sems[0:nbuf]
    gsem = sems[nbuf:2 * nbuf]
    ssem = sems[2 * nbuf:3 * nbuf]
    c = lax.axis_index("c")
    s = lax.axis_index("s")
    sl = pl.ds(s * RPT, RPT)
    pltpu.sync_copy(zeros_hbm.at[sl], acc_sh.at[sl])
    plsc.subcore_barrier()

    def i_descs(j, b):
        return (pltpu.make_async_copy(src_hbm.at[c, s, j], src_v.at[b], isem[b]),
                pltpu.make_async_copy(dst_hbm.at[c, s, j], dst_v.at[b], isem[b]))

    def g_desc(b):
        return pltpu.make_async_copy(
            y_hbm.at[src_v.at[b]], rows_v.at[b], gsem[b])

    def s_desc(b):
        return pltpu.make_async_copy(
            rows_v.at[b], acc_sh.at[dst_v.at[b]], ssem[b])

    def start_idx(j, b):
        d1, d2 = i_descs(j, b)
        d1.start()
        d2.start()

    def wait_idx(j, b):
        d1, d2 = i_descs(j, b)
        d1.wait()
        d2.wait()

    for b in range(nbuf):
        start_idx(b, b)
        wait_idx(b, b)
        g_desc(b).start()

    def body(i, carry):
        j0 = i * nbuf
        for b in range(nbuf):
            g_desc(b).wait()
            s_desc(b).start(add=True)
        for b in range(nbuf):
            s_desc(b).wait()
            start_idx(j0 + nbuf + b, b)
            wait_idx(j0 + nbuf + b, b)
            g_desc(b).start()
        return carry

    lax.fori_loop(0, cpt // nbuf - 1, body, 0)
    for b in range(nbuf):
        g_desc(b).wait()
        s_desc(b).start(add=True)
    for b in range(nbuf):
        s_desc(b).wait()
    plsc.subcore_barrier()
    pltpu.sync_copy(acc_sh.at[sl], out_hbm.at[c, sl])


def _make_agg(D, nbuf, cpt, chunk, tc_tiling):
    return functools.partial(
        pl.kernel,
        out_type=jax.ShapeDtypeStruct((NC, ROWS, D), jnp.float32),
        mesh=_MESH,
        compiler_params=(None if tc_tiling
                         else pltpu.CompilerParams(use_tc_tiling_on_sc=False)),
        scratch_types=[
            pltpu.VMEM((nbuf, chunk), jnp.int32),
            pltpu.VMEM((nbuf, chunk), jnp.int32),
            pltpu.VMEM((nbuf, chunk, D), jnp.float32),
            pltpu.VMEM_SHARED((ROWS, D), jnp.float32),
        ] + [pltpu.SemaphoreType.DMA] * (3 * nbuf),
    )(functools.partial(_sc_agg, D, nbuf, cpt))


_agg128_kernel = _make_agg(D_HID, 2, CPT, CHUNK, True)
_agg64_kernel = _make_agg(D_CLS_PAD, 4, CPT, CHUNK, False)


_BLK = 512
_GRID = ROWS // _BLK


def _dis_block(deg_ref, i):
    d = deg_ref[0, pl.ds(i * _BLK, _BLK)] + deg_ref[1, pl.ds(i * _BLK, _BLK)]
    return lax.rsqrt(d + 1.0)[:, None]


def _row_mask(i):
    rows = i * _BLK + lax.broadcasted_iota(jnp.int32, (_BLK, 1), 0)
    return (rows < N_REAL).astype(jnp.float32)


def _tc_mm1(x_ref, w_ref, deg_ref, y_ref):
    i = pl.program_id(0)
    dis = _dis_block(deg_ref, i)
    y_ref[...] = jnp.dot(x_ref[...], w_ref[...],
                         preferred_element_type=jnp.float32) * dis


def _tc_hidden(agg_ref, y1_ref, deg_ref, b1_ref, w2_ref, y2_ref):
    i = pl.program_id(0)
    dis = _dis_block(deg_ref, i)
    h = (agg_ref[0] + agg_ref[1] + y1_ref[...]) * dis + b1_ref[...]
    h = jnp.maximum(h, 0.0)
    y2_ref[...] = jnp.dot(h, w2_ref[...],
                          preferred_element_type=jnp.float32) * dis * _row_mask(i)


def _tc_out(agg_ref, y2_ref, deg_ref, b2_ref, o_ref):
    i = pl.program_id(0)
    dis = _dis_block(deg_ref, i)
    z = ((agg_ref[0] + agg_ref[1] + y2_ref[...]) * dis)[:, :D_CLS] + b2_ref[...]
    m = jnp.max(z, axis=1, keepdims=True)
    lse = jnp.log(jnp.sum(jnp.exp(z - m), axis=1, keepdims=True)) + m
    o_ref[...] = z - lse


def _mm1_call(x_pad, W1, degp):
    return pl.pallas_call(
        _tc_mm1,
        grid=(_GRID,),
        in_specs=[
            pl.BlockSpec((_BLK, D_IN), lambda i: (i, 0)),
            pl.BlockSpec((D_IN, D_HID), lambda i: (0, 0)),
            pl.BlockSpec((NC, ROWS), lambda i: (0, 0)),
        ],
        out_specs=pl.BlockSpec((_BLK, D_HID), lambda i: (i, 0)),
        out_shape=jax.ShapeDtypeStruct((ROWS, D_HID), jnp.float32),
    )(x_pad, W1, degp)


def _hidden_call(agg1, y1, degp, b1, W2p):
    return pl.pallas_call(
        _tc_hidden,
        grid=(_GRID,),
        in_specs=[
            pl.BlockSpec((NC, _BLK, D_HID), lambda i: (0, i, 0)),
            pl.BlockSpec((_BLK, D_HID), lambda i: (i, 0)),
            pl.BlockSpec((NC, ROWS), lambda i: (0, 0)),
            pl.BlockSpec((1, D_HID), lambda i: (0, 0)),
            pl.BlockSpec((D_HID, D_CLS_PAD), lambda i: (0, 0)),
        ],
        out_specs=pl.BlockSpec((_BLK, D_CLS_PAD), lambda i: (i, 0)),
        out_shape=jax.ShapeDtypeStruct((ROWS, D_CLS_PAD), jnp.float32),
    )(agg1, y1, degp, b1, W2p)


def _out_call(agg2, y2, degp, b2):
    return pl.pallas_call(
        _tc_out,
        grid=(_GRID,),
        in_specs=[
            pl.BlockSpec((NC, _BLK, D_CLS_PAD), lambda i: (0, i, 0)),
            pl.BlockSpec((_BLK, D_CLS_PAD), lambda i: (i, 0)),
            pl.BlockSpec((NC, ROWS), lambda i: (0, 0)),
            pl.BlockSpec((1, D_CLS), lambda i: (0, 0)),
        ],
        out_specs=pl.BlockSpec((_BLK, D_CLS), lambda i: (i, 0)),
        out_shape=jax.ShapeDtypeStruct((ROWS, D_CLS), jnp.float32),
    )(agg2, y2, degp, b2)


def kernel(x, edge_index, W1, b1, W2, b2):
    ei = edge_index.astype(jnp.int32)
    src = ei[0]
    dst = ei[1]
    pad_n = E_PAD - src.shape[0]
    pad_iota = jnp.arange(pad_n, dtype=jnp.int32)
    pad_rows = N_REAL + pad_iota % DUMMY
    srcp = jnp.concatenate([src, pad_rows]).reshape(NC, NS, CPT, CHUNK)
    dstp = jnp.concatenate([dst, pad_rows]).reshape(NC, NS, CPT, CHUNK)

    zeros_deg = jnp.zeros((ROWS,), jnp.float32)
    zeros128 = jnp.zeros((ROWS, D_HID), jnp.float32)
    zeros64 = jnp.zeros((ROWS, D_CLS_PAD), jnp.float32)
    x_pad = jnp.concatenate([x, jnp.zeros((DUMMY, D_IN), jnp.float32)])
    W2p = jnp.concatenate(
        [W2, jnp.zeros((D_HID, D_CLS_PAD - D_CLS), jnp.float32)], axis=1)

    degp = _degree_kernel(dstp, zeros_deg)
    y1 = _mm1_call(x_pad, W1, degp)
    agg1 = _agg128_kernel(srcp, dstp, y1, zeros128)
    y2 = _hidden_call(agg1, y1, degp, b1.reshape(1, D_HID), W2p)
    agg2 = _agg64_kernel(srcp, dstp, y2, zeros64)
    out = _out_call(agg2, y2, degp, b2.reshape(1, D_CLS))
    return out[:N_REAL]


# speedup vs baseline: 35.5720x; 1.1750x over previous
import functools

import jax
import jax.numpy as jnp
from jax import lax
from jax.experimental import pallas as pl
from jax.experimental.pallas import tpu as pltpu
from jax.experimental.pallas import tpu_sc as plsc

N_REAL = 10000
ROWS = 10240
DUMMY = ROWS - N_REAL
NC = 2
NS = 16
CHUNK = 128
CPT = 80
E_PAD = NC * NS * CPT * CHUNK
RPT = ROWS // NS

D_IN = 128
D_HID = 128
D_CLS = 40
D_CLS_PAD = 64

_MESH = plsc.VectorSubcoreMesh(core_axis_name="c", subcore_axis_name="s")


def _sc_degree(dst_hbm, zeros_hbm, out_hbm, dst_v, ones_v, acc_sh):
    c = lax.axis_index("c")
    s = lax.axis_index("s")
    pltpu.sync_copy(dst_hbm.at[c, s], dst_v)
    sl = pl.ds(s * RPT, RPT)
    pltpu.sync_copy(zeros_hbm.at[sl], acc_sh.at[sl])
    for k in range(CHUNK // 16):
        ones_v[pl.ds(k * 16, 16)] = jnp.ones((16,), jnp.float32)
    plsc.subcore_barrier()

    def body(j, carry):
        pltpu.sync_copy(ones_v, acc_sh.at[dst_v.at[j]], add=True)
        return carry

    lax.fori_loop(0, CPT, body, 0)
    plsc.subcore_barrier()
    pltpu.sync_copy(acc_sh.at[sl], out_hbm.at[c, sl])


_degree_kernel = functools.partial(
    pl.kernel,
    out_type=jax.ShapeDtypeStruct((NC, ROWS), jnp.float32),
    mesh=_MESH,
    scratch_types=[
        pltpu.VMEM((CPT, CHUNK), jnp.int32),
        pltpu.VMEM((CHUNK,), jnp.float32),
        pltpu.VMEM_SHARED((ROWS,), jnp.float32),
    ],
)(_sc_degree)


_IRING = 8


def _sc_agg(R, cpt, src_hbm, dst_hbm, y_hbm, zeros_hbm, out_hbm,
            src_v, dst_v, rows_v, acc_sh, *sems):
    isem = sems[0:_IRING]
    gsem = sems[_IRING:_IRING + R]
    ssem = sems[_IRING + R:_IRING + 2 * R]
    c = lax.axis_index("c")
    s = lax.axis_index("s")
    sl = pl.ds(s * RPT, RPT)
    pltpu.sync_copy(zeros_hbm.at[sl], acc_sh.at[sl])
    plsc.subcore_barrier()

    def i_descs(j, k):
        return (pltpu.make_async_copy(src_hbm.at[c, s, j], src_v.at[k], isem[k]),
                pltpu.make_async_copy(dst_hbm.at[c, s, j], dst_v.at[k], isem[k]))

    def g_desc(r, k):
        return pltpu.make_async_copy(
            y_hbm.at[src_v.at[k]], rows_v.at[r], gsem[r])

    def s_desc(r, k):
        return pltpu.make_async_copy(
            rows_v.at[r], acc_sh.at[dst_v.at[k]], ssem[r])

    def start_idx(j, k):
        d1, d2 = i_descs(j, k)
        d1.start()
        d2.start()

    def wait_idx(j, k):
        d1, d2 = i_descs(j, k)
        d1.wait()
        d2.wait()

    for k in range(_IRING):
        start_idx(k, k)
    for r in range(R):
        wait_idx(r, r)
        g_desc(r, r).start()

    def body(i, carry):
        j0 = i * _IRING
        for k in range(_IRING):
            r = k % R
            g_desc(r, k).wait()
            s_desc(r, k).start(add=True)
            s_desc(r, k).wait()
            start_idx(j0 + _IRING + k, k)
            nk = (k + R) % _IRING
            wait_idx(j0 + k + R, nk)
            g_desc(r, nk).start()
        return carry

    lax.fori_loop(0, cpt // _IRING - 1, body, 0)
    tail = cpt - _IRING
    for k in range(_IRING):
        r = k % R
        g_desc(r, k).wait()
        s_desc(r, k).start(add=True)
        s_desc(r, k).wait()
        if k + R < _IRING:
            wait_idx(tail + k + R, k + R)
            g_desc(r, k + R).start()
    plsc.subcore_barrier()
    pltpu.sync_copy(acc_sh.at[sl], out_hbm.at[c, sl])


def _make_agg(D, R, cpt, chunk, tc_tiling):
    return functools.partial(
        pl.kernel,
        out_type=jax.ShapeDtypeStruct((NC, ROWS, D), jnp.float32),
        mesh=_MESH,
        compiler_params=(None if tc_tiling
                         else pltpu.CompilerParams(use_tc_tiling_on_sc=False)),
        scratch_types=[
            pltpu.VMEM((_IRING, chunk), jnp.int32),
            pltpu.VMEM((_IRING, chunk), jnp.int32),
            pltpu.VMEM((R, chunk, D), jnp.float32),
            pltpu.VMEM_SHARED((ROWS, D), jnp.float32),
        ] + [pltpu.SemaphoreType.DMA] * (_IRING + 2 * R),
    )(functools.partial(_sc_agg, R, cpt))


_agg128_kernel = _make_agg(D_HID, 2, CPT, CHUNK, True)
_agg64_kernel = _make_agg(D_CLS_PAD, 4, CPT, CHUNK, False)


_BLK = 512
_GRID = ROWS // _BLK


def _dis_block(deg_ref, i):
    d = deg_ref[0, pl.ds(i * _BLK, _BLK)] + deg_ref[1, pl.ds(i * _BLK, _BLK)]
    return lax.rsqrt(d + 1.0)[:, None]


def _row_mask(i):
    rows = i * _BLK + lax.broadcasted_iota(jnp.int32, (_BLK, 1), 0)
    return (rows < N_REAL).astype(jnp.float32)


def _tc_mm1(x_ref, w_ref, deg_ref, y_ref):
    i = pl.program_id(0)
    dis = _dis_block(deg_ref, i)
    y_ref[...] = jnp.dot(x_ref[...], w_ref[...],
                         preferred_element_type=jnp.float32) * dis


def _tc_hidden(agg_ref, y1_ref, deg_ref, b1_ref, w2_ref, y2_ref):
    i = pl.program_id(0)
    dis = _dis_block(deg_ref, i)
    h = (agg_ref[0] + agg_ref[1] + y1_ref[...]) * dis + b1_ref[...]
    h = jnp.maximum(h, 0.0)
    y2_ref[...] = jnp.dot(h, w2_ref[...],
                          preferred_element_type=jnp.float32) * dis * _row_mask(i)


def _tc_out(agg_ref, y2_ref, deg_ref, b2_ref, o_ref):
    i = pl.program_id(0)
    dis = _dis_block(deg_ref, i)
    z = ((agg_ref[0] + agg_ref[1] + y2_ref[...]) * dis)[:, :D_CLS] + b2_ref[...]
    m = jnp.max(z, axis=1, keepdims=True)
    lse = jnp.log(jnp.sum(jnp.exp(z - m), axis=1, keepdims=True)) + m
    o_ref[...] = z - lse


def _mm1_call(x_pad, W1, degp):
    return pl.pallas_call(
        _tc_mm1,
        grid=(_GRID,),
        in_specs=[
            pl.BlockSpec((_BLK, D_IN), lambda i: (i, 0)),
            pl.BlockSpec((D_IN, D_HID), lambda i: (0, 0)),
            pl.BlockSpec((NC, ROWS), lambda i: (0, 0)),
        ],
        out_specs=pl.BlockSpec((_BLK, D_HID), lambda i: (i, 0)),
        out_shape=jax.ShapeDtypeStruct((ROWS, D_HID), jnp.float32),
    )(x_pad, W1, degp)


def _hidden_call(agg1, y1, degp, b1, W2p):
    return pl.pallas_call(
        _tc_hidden,
        grid=(_GRID,),
        in_specs=[
            pl.BlockSpec((NC, _BLK, D_HID), lambda i: (0, i, 0)),
            pl.BlockSpec((_BLK, D_HID), lambda i: (i, 0)),
            pl.BlockSpec((NC, ROWS), lambda i: (0, 0)),
            pl.BlockSpec((1, D_HID), lambda i: (0, 0)),
            pl.BlockSpec((D_HID, D_CLS_PAD), lambda i: (0, 0)),
        ],
        out_specs=pl.BlockSpec((_BLK, D_CLS_PAD), lambda i: (i, 0)),
        out_shape=jax.ShapeDtypeStruct((ROWS, D_CLS_PAD), jnp.float32),
    )(agg1, y1, degp, b1, W2p)


def _out_call(agg2, y2, degp, b2):
    return pl.pallas_call(
        _tc_out,
        grid=(_GRID,),
        in_specs=[
            pl.BlockSpec((NC, _BLK, D_CLS_PAD), lambda i: (0, i, 0)),
            pl.BlockSpec((_BLK, D_CLS_PAD), lambda i: (i, 0)),
            pl.BlockSpec((NC, ROWS), lambda i: (0, 0)),
            pl.BlockSpec((1, D_CLS), lambda i: (0, 0)),
        ],
        out_specs=pl.BlockSpec((_BLK, D_CLS), lambda i: (i, 0)),
        out_shape=jax.ShapeDtypeStruct((ROWS, D_CLS), jnp.float32),
    )(agg2, y2, degp, b2)


def kernel(x, edge_index, W1, b1, W2, b2):
    ei = edge_index.astype(jnp.int32)
    src = ei[0]
    dst = ei[1]
    pad_n = E_PAD - src.shape[0]
    pad_iota = jnp.arange(pad_n, dtype=jnp.int32)
    pad_rows = N_REAL + pad_iota % DUMMY
    srcp = jnp.concatenate([src, pad_rows]).reshape(NC, NS, CPT, CHUNK)
    dstp = jnp.concatenate([dst, pad_rows]).reshape(NC, NS, CPT, CHUNK)

    zeros_deg = jnp.zeros((ROWS,), jnp.float32)
    zeros128 = jnp.zeros((ROWS, D_HID), jnp.float32)
    zeros64 = jnp.zeros((ROWS, D_CLS_PAD), jnp.float32)
    x_pad = jnp.concatenate([x, jnp.zeros((DUMMY, D_IN), jnp.float32)])
    W2p = jnp.concatenate(
        [W2, jnp.zeros((D_HID, D_CLS_PAD - D_CLS), jnp.float32)], axis=1)

    degp = _degree_kernel(dstp, zeros_deg)
    y1 = _mm1_call(x_pad, W1, degp)
    agg1 = _agg128_kernel(srcp, dstp, y1, zeros128)
    y2 = _hidden_call(agg1, y1, degp, b1.reshape(1, D_HID), W2p)
    agg2 = _agg64_kernel(srcp, dstp, y2, zeros64)
    out = _out_call(agg2, y2, degp, b2.reshape(1, D_CLS))
    return out[:N_REAL]
